# Initial kernel scaffold; baseline (speedup 1.0000x reference)
import jax
import jax.numpy as jnp
from jax.experimental import pallas as pl


def kernel(x, edge_index, W1, b1, p1, W2, b2, p2, W3, b3, Wout, bout):
    raise NotImplementedError("write your pallas kernel here")


# speedup vs baseline: 1.2916x; 1.2916x over previous
import jax
import jax.numpy as jnp
from jax.experimental import pallas as pl


def _gcn_masked(xg, src, dst, alive, W, b, n):
    w = alive[src] * alive[dst]
    deg = jax.ops.segment_sum(w, dst, num_segments=n) + 1.0
    rdeg = jax.lax.rsqrt(deg)
    y = xg @ W
    y2 = y * (rdeg * alive)[:, None]
    S = jax.ops.segment_sum(y2[src], dst, num_segments=n)
    agg = S * rdeg[:, None] + y * (rdeg * rdeg)[:, None]
    return jax.nn.relu(agg + b)


def _pool_masked(h, alive, p, k):
    s = (h @ p) / jnp.linalg.norm(p)
    smask = jnp.where(alive > 0.5, s, -jnp.inf)
    vals = jax.lax.top_k(smask, k)[0]
    thresh = vals[k - 1]
    alive_new = (smask >= thresh).astype(jnp.float32)
    xg = h * jnp.tanh(s)[:, None]
    return xg, alive_new


def kernel(x, edge_index, W1, b1, p1, W2, b2, p2, W3, b3, Wout, bout):
    n = x.shape[0]
    src = edge_index[0]
    dst = edge_index[1]
    alive = jnp.ones((n,), jnp.float32)

    h = _gcn_masked(x, src, dst, alive, W1, b1, n)
    xg, alive = _pool_masked(h, alive, p1, n // 2)
    h = _gcn_masked(xg, src, dst, alive, W2, b2, n)
    xg, alive = _pool_masked(h, alive, p2, n // 4)
    h = _gcn_masked(xg, src, dst, alive, W3, b3, n)

    g = jnp.sum(h * alive[:, None], axis=0) / (n // 4)
    return jax.nn.softmax(g @ Wout + bout)


# speedup vs baseline: 32.8839x; 25.4604x over previous
import functools

import jax
import jax.numpy as jnp
from jax import lax
from jax.experimental import pallas as pl
from jax.experimental.pallas import tpu as pltpu
from jax.experimental.pallas import tpu_sc as plsc

_NC, _NS = 2, 16
_NW = _NC * _NS
_NN = 10000
_NPAD = 10240
_NROW = _NPAD // 128
_EE = 320000
_EPAD = 327680
_EROWS = _EPAD // 128
_RPT = _EROWS // _NW
_TSL = _NPAD // _NS
_F = 128
_BN = 1024
_NB = _NPAD // _BN


def _sc_mesh():
    return plsc.VectorSubcoreMesh(
        core_axis_name="c", subcore_axis_name="s",
        num_cores=_NC, num_subcores=_NS)


def _sc_deg(src2d, dst2d, alive_flat):
    @functools.partial(
        pl.kernel,
        out_type=jax.ShapeDtypeStruct((_NC, _NPAD), jnp.float32),
        mesh=_sc_mesh(),
        compiler_params=pltpu.CompilerParams(needs_layout_passes=False),
        scratch_types=[
            pltpu.VMEM((_RPT, 128), jnp.int32),
            pltpu.VMEM((_RPT, 128), jnp.int32),
            pltpu.VMEM((_NPAD,), jnp.float32),
            pltpu.VMEM((128,), jnp.float32),
            pltpu.VMEM_SHARED((_NPAD,), jnp.float32),
        ],
    )
    def k(src_hbm, dst_hbm, alive_hbm, out_hbm, srcv, dstv, av, wv, acc):
        c = lax.axis_index("c")
        s = lax.axis_index("s")
        w = s * _NC + c
        base = s * _TSL
        for q in range(8):
            wv[pl.ds(q * 16, 16)] = jnp.zeros((16,), jnp.float32)
        for q in range(_TSL // 128):
            pltpu.sync_copy(wv, acc.at[pl.ds(base + q * 128, 128)])
        pltpu.sync_copy(alive_hbm, av)
        pltpu.sync_copy(src_hbm.at[pl.ds(w * _RPT, _RPT)], srcv)
        pltpu.sync_copy(dst_hbm.at[pl.ds(w * _RPT, _RPT)], dstv)
        plsc.subcore_barrier()

        def body(j, carry):
            for q in range(8):
                sv = srcv[j, pl.ds(q * 16, 16)]
                dv = dstv[j, pl.ds(q * 16, 16)]
                a = plsc.load_gather(av, [sv]) * plsc.load_gather(av, [dv])
                wv[pl.ds(q * 16, 16)] = a
            pltpu.sync_copy(wv, acc.at[dstv.at[j]], add=True)
            return carry

        lax.fori_loop(0, _RPT, body, 0)
        plsc.subcore_barrier()
        pltpu.sync_copy(acc.at[pl.ds(base, _TSL)],
                        out_hbm.at[c, pl.ds(base, _TSL)])

    return k(src2d, dst2d, alive_flat)


def _sc_agg(src2d, dst2d, y2, zinit):
    @functools.partial(
        pl.kernel,
        out_type=jax.ShapeDtypeStruct((_NC, _NPAD, _F), jnp.float32),
        mesh=_sc_mesh(),
        compiler_params=pltpu.CompilerParams(needs_layout_passes=False),
        scratch_types=[
            pltpu.VMEM((_RPT, 128), jnp.int32),
            pltpu.VMEM((_RPT, 128), jnp.int32),
            pltpu.VMEM((128, _F), jnp.float32),
            pltpu.VMEM_SHARED((_NPAD, _F), jnp.float32),
        ],
    )
    def k(src_hbm, dst_hbm, y2_hbm, z_hbm, out_hbm, srcv, dstv, rows, acc):
        c = lax.axis_index("c")
        s = lax.axis_index("s")
        w = s * _NC + c
        base = s * _TSL

        @pl.when(c == 0)
        def _():
            pltpu.sync_copy(y2_hbm.at[pl.ds(base, _TSL)],
                            acc.at[pl.ds(base, _TSL)])

        @pl.when(c != 0)
        def _():
            pltpu.sync_copy(z_hbm.at[pl.ds(base, _TSL)],
                            acc.at[pl.ds(base, _TSL)])

        pltpu.sync_copy(src_hbm.at[pl.ds(w * _RPT, _RPT)], srcv)
        pltpu.sync_copy(dst_hbm.at[pl.ds(w * _RPT, _RPT)], dstv)
        plsc.subcore_barrier()

        def body(j, carry):
            pltpu.sync_copy(y2_hbm.at[srcv.at[j]], rows)
            pltpu.sync_copy(rows, acc.at[dstv.at[j]], add=True)
            return carry

        lax.fori_loop(0, _RPT, body, 0)
        plsc.subcore_barrier()
        pltpu.sync_copy(acc.at[pl.ds(base, _TSL)],
                        out_hbm.at[c, pl.ds(base, _TSL)])

    return k(src2d, dst2d, y2, zinit)


def _tc_scale(xg, W, degp, alive_col):
    def body(x_ref, w_ref, deg_ref, alive_ref, y2_ref, rdeg_ref):
        y = jnp.dot(x_ref[...], w_ref[...],
                    preferred_element_type=jnp.float32)
        d = deg_ref[...]
        rdeg = lax.rsqrt(d[0] + d[1] + 1.0)
        y2_ref[...] = y * (rdeg * alive_ref[...])
        rdeg_ref[...] = rdeg

    return pl.pallas_call(
        body,
        grid=(_NB,),
        in_specs=[
            pl.BlockSpec((_BN, _F), lambda i: (i, 0)),
            pl.BlockSpec((_F, _F), lambda i: (0, 0)),
            pl.BlockSpec((_NC, _BN, 1), lambda i: (0, i, 0)),
            pl.BlockSpec((_BN, 1), lambda i: (i, 0)),
        ],
        out_specs=[
            pl.BlockSpec((_BN, _F), lambda i: (i, 0)),
            pl.BlockSpec((_BN, 1), lambda i: (i, 0)),
        ],
        out_shape=[
            jax.ShapeDtypeStruct((_NPAD, _F), jnp.float32),
            jax.ShapeDtypeStruct((_NPAD, 1), jnp.float32),
        ],
    )(xg, W, degp, alive_col)


def _tc_pool_a(S, rdeg, b2d, p2d):
    def body(s_ref, rdeg_ref, b_ref, p_ref, xg_ref, sc_ref):
        p = p_ref[...]
        pn = lax.rsqrt(jnp.sum(p * p))
        st = s_ref[...]
        h = jnp.maximum((st[0] + st[1]) * rdeg_ref[...] + b_ref[...], 0.0)
        sc = jnp.sum(h * p, axis=1, keepdims=True) * pn
        xg_ref[...] = h * jnp.tanh(sc)
        sc_ref[...] = sc

    return pl.pallas_call(
        body,
        grid=(_NB,),
        in_specs=[
            pl.BlockSpec((_NC, _BN, _F), lambda i: (0, i, 0)),
            pl.BlockSpec((_BN, 1), lambda i: (i, 0)),
            pl.BlockSpec((1, _F), lambda i: (0, 0)),
            pl.BlockSpec((1, _F), lambda i: (0, 0)),
        ],
        out_specs=[
            pl.BlockSpec((_BN, _F), lambda i: (i, 0)),
            pl.BlockSpec((_BN, 1), lambda i: (i, 0)),
        ],
        out_shape=[
            jax.ShapeDtypeStruct((_NPAD, _F), jnp.float32),
            jax.ShapeDtypeStruct((_NPAD, 1), jnp.float32),
        ],
    )(S, rdeg, b2d, p2d)


def _tc_pool_b(scores2d, alive2d, kkeep):
    def body(sc_ref, alive_ref, out_ref):
        sc = sc_ref[...]
        bits = lax.bitcast_convert_type(sc, jnp.int32)
        ubits = lax.bitcast_convert_type(sc, jnp.uint32)
        flip = jnp.where(bits < 0, jnp.uint32(0xFFFFFFFF),
                         jnp.uint32(0x80000000))
        keys = jnp.where(alive_ref[...] > 0.5, ubits ^ flip, jnp.uint32(0))

        def bit(t, T):
            cand = T | (jnp.uint32(1) << (jnp.uint32(31) - t.astype(jnp.uint32)))
            cnt = jnp.sum((keys >= cand).astype(jnp.int32))
            return jnp.where(cnt >= kkeep, cand, T)

        T = lax.fori_loop(0, 32, bit, jnp.uint32(0))
        out_ref[...] = (keys >= T).astype(jnp.float32)

    return pl.pallas_call(
        body,
        out_shape=jax.ShapeDtypeStruct((_NROW, 128), jnp.float32),
    )(scores2d, alive2d)


def _tc_final(S, rdeg, b2d, alive_col, WoutP, boutP):
    def body(s_ref, rdeg_ref, b_ref, alive_ref, wo_ref, bo_ref, out_ref):
        def blk(i, g):
            sl = pl.ds(i * _BN, _BN)
            st = s_ref[0, sl, :] + s_ref[1, sl, :]
            h = jnp.maximum(st * rdeg_ref[sl, :] + b_ref[...], 0.0)
            return g + jnp.sum(h * alive_ref[sl, :], axis=0, keepdims=True)

        g = lax.fori_loop(0, _NB, blk, jnp.zeros((1, _F), jnp.float32))
        g = g * (1.0 / 2500.0)
        z = jnp.dot(g, wo_ref[...], preferred_element_type=jnp.float32)
        z = z + bo_ref[...]
        col = lax.broadcasted_iota(jnp.int32, (1, _F), 1)
        z = jnp.where(col < 2, z, -1e30)
        m = jnp.max(z)
        e = jnp.exp(z - m)
        out_ref[...] = e / jnp.sum(e)

    return pl.pallas_call(
        body,
        out_shape=jax.ShapeDtypeStruct((1, _F), jnp.float32),
    )(S, rdeg, b2d, alive_col, WoutP, boutP)


def kernel(x, edge_index, W1, b1, p1, W2, b2, p2, W3, b3, Wout, bout):
    f32 = jnp.float32
    src = edge_index[0]
    dst = edge_index[1]
    npadrows = _NPAD - _NN
    padi = _NN + (jnp.arange(_EPAD - _EE, dtype=jnp.int32) % npadrows)
    src2d = jnp.concatenate([src, padi]).reshape(_EROWS, 128)
    dst2d = jnp.concatenate([dst, padi]).reshape(_EROWS, 128)
    xp = jnp.pad(x, ((0, npadrows), (0, 0)))
    zinit = jnp.zeros((_NPAD, _F), f32)
    alive = (jnp.arange(_NPAD, dtype=jnp.int32) < _NN).astype(f32)
    alive = alive.reshape(_NROW, 128)
    WoutP = jnp.pad(Wout, ((0, 0), (0, _F - Wout.shape[1])))
    boutP = jnp.pad(bout, (0, _F - bout.shape[0])).reshape(1, _F)

    xg = xp
    for (W, b, p, kkeep) in ((W1, b1, p1, _NN // 2), (W2, b2, p2, _NN // 4)):
        degp = _sc_deg(src2d, dst2d, alive.reshape(_NPAD))
        y2, rdeg = _tc_scale(xg, W, degp.reshape(_NC, _NPAD, 1),
                             alive.reshape(_NPAD, 1))
        S = _sc_agg(src2d, dst2d, y2, zinit)
        xg, scores = _tc_pool_a(S, rdeg, b.reshape(1, _F), p.reshape(1, _F))
        alive = _tc_pool_b(scores.reshape(_NROW, 128), alive, kkeep)

    degp = _sc_deg(src2d, dst2d, alive.reshape(_NPAD))
    y2, rdeg = _tc_scale(xg, W3, degp.reshape(_NC, _NPAD, 1),
                         alive.reshape(_NPAD, 1))
    S = _sc_agg(src2d, dst2d, y2, zinit)
    probs = _tc_final(S, rdeg, b3.reshape(1, _F), alive.reshape(_NPAD, 1),
                      WoutP, boutP)
    return probs[0, :2]


# speedup vs baseline: 50.5216x; 1.5364x over previous
import functools

import jax
import jax.numpy as jnp
from jax import lax
from jax.experimental import pallas as pl
from jax.experimental.pallas import tpu as pltpu
from jax.experimental.pallas import tpu_sc as plsc

_NC, _NS = 2, 16
_NW = _NC * _NS
_NN = 10000
_NPAD = 10240
_NROW = _NPAD // 128
_EE = 320000
_EPAD = 327680
_EROWS = _EPAD // 128
_RPT = _EROWS // _NW
_TSL = _NPAD // _NS
_CAP = 88
_CAPW = _CAP * 128
_F = 128
_BN = 1024
_NB = _NPAD // _BN


def _sc_mesh():
    return plsc.VectorSubcoreMesh(
        core_axis_name="c", subcore_axis_name="s",
        num_cores=_NC, num_subcores=_NS)


def _sc_degc(esrc, edst, ecnt, alive_flat):
    rin = esrc.shape[0] // _NW
    has_cnt = ecnt is not None

    def k(*args):
        if has_cnt:
            (esrc_hbm, edst_hbm, ecnt_hbm, alive_hbm, deg_hbm, csrc_hbm,
             cdst_hbm, ccnt_hbm, srcv, dstv, av, wv, csrcv, cdstv, cntv,
             acc) = args
        else:
            (esrc_hbm, edst_hbm, alive_hbm, deg_hbm, csrc_hbm,
             cdst_hbm, ccnt_hbm, srcv, dstv, av, wv, csrcv, cdstv, cntv,
             acc) = args
        c = lax.axis_index("c")
        s = lax.axis_index("s")
        w = s * _NC + c
        base = s * _TSL
        for q in range(8):
            wv[pl.ds(q * 16, 16)] = jnp.zeros((16,), jnp.float32)
        for q in range(_TSL // 128):
            pltpu.sync_copy(wv, acc.at[pl.ds(base + q * 128, 128)])
        pltpu.sync_copy(alive_hbm, av)
        pltpu.sync_copy(esrc_hbm.at[pl.ds(w * rin, rin)], srcv)
        pltpu.sync_copy(edst_hbm.at[pl.ds(w * rin, rin)], dstv)
        if has_cnt:
            pltpu.sync_copy(ecnt_hbm.at[w], cntv)
        plsc.subcore_barrier()
        cin = cntv[0, pl.ds(0, 16)][0] if has_cnt else rin

        def body(j, cur):
            for q in range(8):
                sv = srcv[j, pl.ds(q * 16, 16)]
                dv = dstv[j, pl.ds(q * 16, 16)]
                a = plsc.load_gather(av, [sv]) * plsc.load_gather(av, [dv])
                wv[pl.ds(q * 16, 16)] = a
                m = a > 0.0
                plsc.store_compressed(csrcv.at[pl.ds(cur, 16)], sv, mask=m)
                plsc.store_compressed(cdstv.at[pl.ds(cur, 16)], dv, mask=m)
                cur = cur + jnp.sum(m.astype(jnp.int32))
            pltpu.sync_copy(wv, acc.at[dstv.at[j]], add=True)
            return cur

        cur = lax.fori_loop(0, cin, body, jnp.int32(0))
        ii = lax.iota(jnp.int32, 16)
        for q in range(8):
            pv = _NN + ((ii * 8 + q) % (_NPAD - _NN))
            csrcv[pl.ds(cur + q * 16, 16)] = pv
            cdstv[pl.ds(cur + q * 16, 16)] = pv
        nch = (cur + 127) // 128
        cntv[0, :] = jnp.zeros((16,), jnp.int32) + nch
        pltpu.sync_copy(cntv, ccnt_hbm.at[w])
        pltpu.sync_copy(csrcv, csrc_hbm.at[pl.ds(w * _CAPW, _CAPW)])
        pltpu.sync_copy(cdstv, cdst_hbm.at[pl.ds(w * _CAPW, _CAPW)])
        plsc.subcore_barrier()
        pltpu.sync_copy(acc.at[pl.ds(base, _TSL)],
                        deg_hbm.at[c, pl.ds(base, _TSL)])

    kk = functools.partial(
        pl.kernel,
        out_type=[
            jax.ShapeDtypeStruct((_NC, _NPAD), jnp.float32),
            jax.ShapeDtypeStruct((_NW * _CAPW,), jnp.int32),
            jax.ShapeDtypeStruct((_NW * _CAPW,), jnp.int32),
            jax.ShapeDtypeStruct((_NW, 1, 16), jnp.int32),
        ],
        mesh=_sc_mesh(),
        compiler_params=pltpu.CompilerParams(needs_layout_passes=False),
        scratch_types=[
            pltpu.VMEM((rin, 128), jnp.int32),
            pltpu.VMEM((rin, 128), jnp.int32),
            pltpu.VMEM((_NPAD,), jnp.float32),
            pltpu.VMEM((128,), jnp.float32),
            pltpu.VMEM((_CAPW,), jnp.int32),
            pltpu.VMEM((_CAPW,), jnp.int32),
            pltpu.VMEM((1, 16), jnp.int32),
            pltpu.VMEM_SHARED((_NPAD,), jnp.float32),
        ],
    )(k)
    if has_cnt:
        return kk(esrc, edst, ecnt, alive_flat)
    return kk(esrc, edst, alive_flat)


def _sc_agg(src2d, dst2d, ccnt, y2, zinit):
    @functools.partial(
        pl.kernel,
        out_type=jax.ShapeDtypeStruct((_NC, _NPAD, _F), jnp.float32),
        mesh=_sc_mesh(),
        compiler_params=pltpu.CompilerParams(needs_layout_passes=False),
        scratch_types=[
            pltpu.VMEM((_CAP, 128), jnp.int32),
            pltpu.VMEM((_CAP, 128), jnp.int32),
            pltpu.VMEM((1, 16), jnp.int32),
            pltpu.VMEM((128, _F), jnp.float32),
            pltpu.VMEM_SHARED((_NPAD, _F), jnp.float32),
        ],
    )
    def k(src_hbm, dst_hbm, ccnt_hbm, y2_hbm, z_hbm, out_hbm, srcv, dstv,
          cntv, rows, acc):
        c = lax.axis_index("c")
        s = lax.axis_index("s")
        w = s * _NC + c
        base = s * _TSL

        @pl.when(c == 0)
        def _():
            pltpu.sync_copy(y2_hbm.at[pl.ds(base, _TSL)],
                            acc.at[pl.ds(base, _TSL)])

        @pl.when(c != 0)
        def _():
            pltpu.sync_copy(z_hbm.at[pl.ds(base, _TSL)],
                            acc.at[pl.ds(base, _TSL)])

        plsc.subcore_barrier()


        def fire_s(j, b):
            pltpu.async_copy(rows.at[b], acc.at[dstv.at[j]], sems[b],
                             add=True)

        def wait_s(b):
            pltpu.make_async_copy(rows.at[b], acc.at[dstv.at[0]],
                                  sems[b]).wait()

        pltpu.sync_copy(src_hbm.at[pl.ds(w * _CAP, _CAP)], srcv)
        pltpu.sync_copy(dst_hbm.at[pl.ds(w * _CAP, _CAP)], dstv)
        pltpu.sync_copy(ccnt_hbm.at[w], cntv)

        def body(j, carry):
            pltpu.sync_copy(y2_hbm.at[srcv.at[j]], rows)
            pltpu.sync_copy(rows, acc.at[dstv.at[j]], add=True)
            return carry

        lax.fori_loop(0, cntv[0, pl.ds(0, 16)][0], body, 0)
        plsc.subcore_barrier()
        pltpu.sync_copy(acc.at[pl.ds(base, _TSL)],
                        out_hbm.at[c, pl.ds(base, _TSL)])

    return k(src2d, dst2d, ccnt, y2, zinit)


def _tc_scale(xg, W, degp, alive_col):
    def body(x_ref, w_ref, deg_ref, alive_ref, y2_ref, rdeg_ref):
        y = jnp.dot(x_ref[...], w_ref[...],
                    preferred_element_type=jnp.float32)
        d = deg_ref[...]
        rdeg = lax.rsqrt(d[0] + d[1] + 1.0)
        y2_ref[...] = y * (rdeg * alive_ref[...])
        rdeg_ref[...] = rdeg

    return pl.pallas_call(
        body,
        grid=(_NB,),
        in_specs=[
            pl.BlockSpec((_BN, _F), lambda i: (i, 0)),
            pl.BlockSpec((_F, _F), lambda i: (0, 0)),
            pl.BlockSpec((_NC, _BN, 1), lambda i: (0, i, 0)),
            pl.BlockSpec((_BN, 1), lambda i: (i, 0)),
        ],
        out_specs=[
            pl.BlockSpec((_BN, _F), lambda i: (i, 0)),
            pl.BlockSpec((_BN, 1), lambda i: (i, 0)),
        ],
        out_shape=[
            jax.ShapeDtypeStruct((_NPAD, _F), jnp.float32),
            jax.ShapeDtypeStruct((_NPAD, 1), jnp.float32),
        ],
    )(xg, W, degp, alive_col)


def _tc_pool_a(S, rdeg, b2d, p2d):
    def body(s_ref, rdeg_ref, b_ref, p_ref, xg_ref, sc_ref):
        p = p_ref[...]
        pn = lax.rsqrt(jnp.sum(p * p))
        st = s_ref[...]
        h = jnp.maximum((st[0] + st[1]) * rdeg_ref[...] + b_ref[...], 0.0)
        sc = jnp.sum(h * p, axis=1, keepdims=True) * pn
        xg_ref[...] = h * jnp.tanh(sc)
        sc_ref[...] = sc

    return pl.pallas_call(
        body,
        grid=(_NB,),
        in_specs=[
            pl.BlockSpec((_NC, _BN, _F), lambda i: (0, i, 0)),
            pl.BlockSpec((_BN, 1), lambda i: (i, 0)),
            pl.BlockSpec((1, _F), lambda i: (0, 0)),
            pl.BlockSpec((1, _F), lambda i: (0, 0)),
        ],
        out_specs=[
            pl.BlockSpec((_BN, _F), lambda i: (i, 0)),
            pl.BlockSpec((_BN, 1), lambda i: (i, 0)),
        ],
        out_shape=[
            jax.ShapeDtypeStruct((_NPAD, _F), jnp.float32),
            jax.ShapeDtypeStruct((_NPAD, 1), jnp.float32),
        ],
    )(S, rdeg, b2d, p2d)


def _tc_pool_b(scores2d, alive2d, kkeep):
    def body(sc_ref, alive_ref, out_ref):
        sc = sc_ref[...]
        bits = lax.bitcast_convert_type(sc, jnp.int32)
        ubits = lax.bitcast_convert_type(sc, jnp.uint32)
        flip = jnp.where(bits < 0, jnp.uint32(0xFFFFFFFF),
                         jnp.uint32(0x80000000))
        keys = jnp.where(alive_ref[...] > 0.5, ubits ^ flip, jnp.uint32(0))

        def bit(t, T):
            cand = T | (jnp.uint32(1) << (jnp.uint32(31) - t.astype(jnp.uint32)))
            cnt = jnp.sum((keys >= cand).astype(jnp.int32))
            return jnp.where(cnt >= kkeep, cand, T)

        T = lax.fori_loop(0, 32, bit, jnp.uint32(0))
        out_ref[...] = (keys >= T).astype(jnp.float32)

    return pl.pallas_call(
        body,
        out_shape=jax.ShapeDtypeStruct((_NROW, 128), jnp.float32),
    )(scores2d, alive2d)


def _tc_final(S, rdeg, b2d, alive_col, WoutP, boutP):
    def body(s_ref, rdeg_ref, b_ref, alive_ref, wo_ref, bo_ref, out_ref):
        def blk(i, g):
            sl = pl.ds(i * _BN, _BN)
            st = s_ref[0, sl, :] + s_ref[1, sl, :]
            h = jnp.maximum(st * rdeg_ref[sl, :] + b_ref[...], 0.0)
            return g + jnp.sum(h * alive_ref[sl, :], axis=0, keepdims=True)

        g = lax.fori_loop(0, _NB, blk, jnp.zeros((1, _F), jnp.float32))
        g = g * (1.0 / 2500.0)
        z = jnp.dot(g, wo_ref[...], preferred_element_type=jnp.float32)
        z = z + bo_ref[...]
        col = lax.broadcasted_iota(jnp.int32, (1, _F), 1)
        z = jnp.where(col < 2, z, -1e30)
        m = jnp.max(z)
        e = jnp.exp(z - m)
        out_ref[...] = e / jnp.sum(e)

    return pl.pallas_call(
        body,
        out_shape=jax.ShapeDtypeStruct((1, _F), jnp.float32),
    )(S, rdeg, b2d, alive_col, WoutP, boutP)


def kernel(x, edge_index, W1, b1, p1, W2, b2, p2, W3, b3, Wout, bout):
    f32 = jnp.float32
    src = edge_index[0]
    dst = edge_index[1]
    npadrows = _NPAD - _NN
    padi = _NN + (jnp.arange(_EPAD - _EE, dtype=jnp.int32) % npadrows)
    src2d = jnp.concatenate([src, padi]).reshape(_EROWS, 128)
    dst2d = jnp.concatenate([dst, padi]).reshape(_EROWS, 128)
    xp = jnp.pad(x, ((0, npadrows), (0, 0)))
    zinit = jnp.zeros((_NPAD, _F), f32)
    alive = (jnp.arange(_NPAD, dtype=jnp.int32) < _NN).astype(f32)
    alive = alive.reshape(_NROW, 128)
    WoutP = jnp.pad(Wout, ((0, 0), (0, _F - Wout.shape[1])))
    boutP = jnp.pad(bout, (0, _F - bout.shape[0])).reshape(1, _F)

    xg = xp
    esrc, edst, ecnt = src2d, dst2d, None
    for (W, b, p, kkeep) in ((W1, b1, p1, _NN // 2), (W2, b2, p2, _NN // 4)):
        degp, csrc, cdst, ecnt = _sc_degc(esrc, edst, ecnt,
                                          alive.reshape(_NPAD))
        esrc = csrc.reshape(_NW * _CAP, 128)
        edst = cdst.reshape(_NW * _CAP, 128)
        y2, rdeg = _tc_scale(xg, W, degp.reshape(_NC, _NPAD, 1),
                             alive.reshape(_NPAD, 1))
        S = _sc_agg(esrc, edst, ecnt, y2, zinit)
        xg, scores = _tc_pool_a(S, rdeg, b.reshape(1, _F), p.reshape(1, _F))
        alive = _tc_pool_b(scores.reshape(_NROW, 128), alive, kkeep)

    degp, csrc, cdst, ecnt = _sc_degc(esrc, edst, ecnt, alive.reshape(_NPAD))
    esrc = csrc.reshape(_NW * _CAP, 128)
    edst = cdst.reshape(_NW * _CAP, 128)
    y2, rdeg = _tc_scale(xg, W3, degp.reshape(_NC, _NPAD, 1),
                         alive.reshape(_NPAD, 1))
    S = _sc_agg(esrc, edst, ecnt, y2, zinit)
    probs = _tc_final(S, rdeg, b3.reshape(1, _F), alive.reshape(_NPAD, 1),
                      WoutP, boutP)
    return probs[0, :2]


# speedup vs baseline: 50.6611x; 1.0028x over previous
import functools

import jax
import jax.numpy as jnp
from jax import lax
from jax.experimental import pallas as pl
from jax.experimental.pallas import tpu as pltpu
from jax.experimental.pallas import tpu_sc as plsc

_NC, _NS = 2, 16
_NW = _NC * _NS
_NN = 10000
_NPAD = 10240
_NROW = _NPAD // 128
_EE = 320000
_EPAD = 327680
_EROWS = _EPAD // 128
_RPT = _EROWS // _NW
_TSL = _NPAD // _NS
_CAP = 88
_CAPW = _CAP * 128
_F = 128
_BN = 1024
_NB = _NPAD // _BN


def _sc_mesh():
    return plsc.VectorSubcoreMesh(
        core_axis_name="c", subcore_axis_name="s",
        num_cores=_NC, num_subcores=_NS)


def _sc_degc(esrc, edst, ecnt, alive_flat):
    rin = esrc.shape[0] // _NW
    has_cnt = ecnt is not None

    def k(*args):
        if has_cnt:
            (esrc_hbm, edst_hbm, ecnt_hbm, alive_hbm, deg_hbm, csrc_hbm,
             cdst_hbm, ccnt_hbm, srcv, dstv, av, wv, csrcv, cdstv, cntv,
             acc) = args
        else:
            (esrc_hbm, edst_hbm, alive_hbm, deg_hbm, csrc_hbm,
             cdst_hbm, ccnt_hbm, srcv, dstv, av, wv, csrcv, cdstv, cntv,
             acc) = args
        c = lax.axis_index("c")
        s = lax.axis_index("s")
        w = s * _NC + c
        base = s * _TSL
        for q in range(8):
            wv[pl.ds(q * 16, 16)] = jnp.zeros((16,), jnp.float32)
        for q in range(_TSL // 128):
            pltpu.sync_copy(wv, acc.at[pl.ds(base + q * 128, 128)])
        pltpu.sync_copy(alive_hbm, av)
        pltpu.sync_copy(esrc_hbm.at[pl.ds(w * rin, rin)], srcv)
        pltpu.sync_copy(edst_hbm.at[pl.ds(w * rin, rin)], dstv)
        if has_cnt:
            pltpu.sync_copy(ecnt_hbm.at[w], cntv)
        plsc.subcore_barrier()
        cin = cntv[0, pl.ds(0, 16)][0] if has_cnt else rin

        def body(j, cur):
            for q in range(8):
                sv = srcv[j, pl.ds(q * 16, 16)]
                dv = dstv[j, pl.ds(q * 16, 16)]
                a = plsc.load_gather(av, [sv]) * plsc.load_gather(av, [dv])
                wv[pl.ds(q * 16, 16)] = a
                m = a > 0.0
                plsc.store_compressed(csrcv.at[pl.ds(cur, 16)], sv, mask=m)
                plsc.store_compressed(cdstv.at[pl.ds(cur, 16)], dv, mask=m)
                cur = cur + jnp.sum(m.astype(jnp.int32))
            pltpu.sync_copy(wv, acc.at[dstv.at[j]], add=True)
            return cur

        cur = lax.fori_loop(0, cin, body, jnp.int32(0))
        ii = lax.iota(jnp.int32, 16)
        for q in range(8):
            pv = _NN + ((ii * 8 + q) % (_NPAD - _NN))
            csrcv[pl.ds(cur + q * 16, 16)] = pv
            cdstv[pl.ds(cur + q * 16, 16)] = pv
        nch = (cur + 127) // 128
        cntv[0, :] = jnp.zeros((16,), jnp.int32) + nch
        pltpu.sync_copy(cntv, ccnt_hbm.at[w])
        pltpu.sync_copy(csrcv, csrc_hbm.at[pl.ds(w * _CAPW, _CAPW)])
        pltpu.sync_copy(cdstv, cdst_hbm.at[pl.ds(w * _CAPW, _CAPW)])
        plsc.subcore_barrier()
        pltpu.sync_copy(acc.at[pl.ds(base, _TSL)],
                        deg_hbm.at[c, pl.ds(base, _TSL)])

    kk = functools.partial(
        pl.kernel,
        out_type=[
            jax.ShapeDtypeStruct((_NC, _NPAD), jnp.float32),
            jax.ShapeDtypeStruct((_NW * _CAPW,), jnp.int32),
            jax.ShapeDtypeStruct((_NW * _CAPW,), jnp.int32),
            jax.ShapeDtypeStruct((_NW, 1, 16), jnp.int32),
        ],
        mesh=_sc_mesh(),
        compiler_params=pltpu.CompilerParams(needs_layout_passes=False),
        scratch_types=[
            pltpu.VMEM((rin, 128), jnp.int32),
            pltpu.VMEM((rin, 128), jnp.int32),
            pltpu.VMEM((_NPAD,), jnp.float32),
            pltpu.VMEM((128,), jnp.float32),
            pltpu.VMEM((_CAPW,), jnp.int32),
            pltpu.VMEM((_CAPW,), jnp.int32),
            pltpu.VMEM((1, 16), jnp.int32),
            pltpu.VMEM_SHARED((_NPAD,), jnp.float32),
        ],
    )(k)
    if has_cnt:
        return kk(esrc, edst, ecnt, alive_flat)
    return kk(esrc, edst, alive_flat)


def _sc_agg(src2d, dst2d, ccnt, y2):
    @functools.partial(
        pl.kernel,
        out_type=jax.ShapeDtypeStruct((_NC, _NPAD, _F), jnp.float32),
        mesh=_sc_mesh(),
        compiler_params=pltpu.CompilerParams(needs_layout_passes=False),
        scratch_types=[
            pltpu.VMEM((_CAP, 128), jnp.int32),
            pltpu.VMEM((_CAP, 128), jnp.int32),
            pltpu.VMEM((1, 16), jnp.int32),
            pltpu.VMEM((128, _F), jnp.float32),
            pltpu.VMEM_SHARED((_NPAD, _F), jnp.float32),
        ],
    )
    def k(src_hbm, dst_hbm, ccnt_hbm, y2_hbm, out_hbm, srcv, dstv,
          cntv, rows, acc):
        c = lax.axis_index("c")
        s = lax.axis_index("s")
        w = s * _NC + c
        base = s * _TSL

        @pl.when(c == 0)
        def _():
            pltpu.sync_copy(y2_hbm.at[pl.ds(base, _TSL)],
                            acc.at[pl.ds(base, _TSL)])

        @pl.when(c != 0)
        def _():
            def zbody(r, carry):
                for q in range(8):
                    rows[r, pl.ds(q * 16, 16)] = jnp.zeros((16,),
                                                           jnp.float32)
                return carry

            lax.fori_loop(0, 128, zbody, 0)
            for t in range(_TSL // 128):
                pltpu.sync_copy(rows, acc.at[pl.ds(base + t * 128, 128)])

        plsc.subcore_barrier()


        def fire_s(j, b):
            pltpu.async_copy(rows.at[b], acc.at[dstv.at[j]], sems[b],
                             add=True)

        def wait_s(b):
            pltpu.make_async_copy(rows.at[b], acc.at[dstv.at[0]],
                                  sems[b]).wait()

        pltpu.sync_copy(src_hbm.at[pl.ds(w * _CAP, _CAP)], srcv)
        pltpu.sync_copy(dst_hbm.at[pl.ds(w * _CAP, _CAP)], dstv)
        pltpu.sync_copy(ccnt_hbm.at[w], cntv)

        def body(j, carry):
            pltpu.sync_copy(y2_hbm.at[srcv.at[j]], rows)
            pltpu.sync_copy(rows, acc.at[dstv.at[j]], add=True)
            return carry

        lax.fori_loop(0, cntv[0, pl.ds(0, 16)][0], body, 0)
        plsc.subcore_barrier()
        pltpu.sync_copy(acc.at[pl.ds(base, _TSL)],
                        out_hbm.at[c, pl.ds(base, _TSL)])

    return k(src2d, dst2d, ccnt, y2)


def _tc_matmul(xg, W):
    def body(x_ref, w_ref, y_ref):
        y_ref[...] = jnp.dot(x_ref[...], w_ref[...],
                             preferred_element_type=jnp.float32)

    return pl.pallas_call(
        body,
        grid=(_NB,),
        in_specs=[
            pl.BlockSpec((_BN, _F), lambda i: (i, 0)),
            pl.BlockSpec((_F, _F), lambda i: (0, 0)),
        ],
        out_specs=pl.BlockSpec((_BN, _F), lambda i: (i, 0)),
        out_shape=jax.ShapeDtypeStruct((_NPAD, _F), jnp.float32),
    )(xg, W)


def _tc_scale(y, degp, alive_col):
    def body(y_ref, deg_ref, alive_ref, y2_ref, rdeg_ref):
        d = deg_ref[...]
        rdeg = lax.rsqrt(d[0] + d[1] + 1.0)
        y2_ref[...] = y_ref[...] * (rdeg * alive_ref[...])
        rdeg_ref[...] = rdeg

    return pl.pallas_call(
        body,
        grid=(_NB,),
        in_specs=[
            pl.BlockSpec((_BN, _F), lambda i: (i, 0)),
            pl.BlockSpec((_NC, _BN, 1), lambda i: (0, i, 0)),
            pl.BlockSpec((_BN, 1), lambda i: (i, 0)),
        ],
        out_specs=[
            pl.BlockSpec((_BN, _F), lambda i: (i, 0)),
            pl.BlockSpec((_BN, 1), lambda i: (i, 0)),
        ],
        out_shape=[
            jax.ShapeDtypeStruct((_NPAD, _F), jnp.float32),
            jax.ShapeDtypeStruct((_NPAD, 1), jnp.float32),
        ],
    )(y, degp, alive_col)


def _tc_pool_a(S, rdeg, b2d, p2d):
    def body(s_ref, rdeg_ref, b_ref, p_ref, xg_ref, sc_ref):
        p = p_ref[...]
        pn = lax.rsqrt(jnp.sum(p * p))
        st = s_ref[...]
        h = jnp.maximum((st[0] + st[1]) * rdeg_ref[...] + b_ref[...], 0.0)
        sc = jnp.sum(h * p, axis=1, keepdims=True) * pn
        xg_ref[...] = h * jnp.tanh(sc)
        sc_ref[...] = sc

    return pl.pallas_call(
        body,
        grid=(_NB,),
        in_specs=[
            pl.BlockSpec((_NC, _BN, _F), lambda i: (0, i, 0)),
            pl.BlockSpec((_BN, 1), lambda i: (i, 0)),
            pl.BlockSpec((1, _F), lambda i: (0, 0)),
            pl.BlockSpec((1, _F), lambda i: (0, 0)),
        ],
        out_specs=[
            pl.BlockSpec((_BN, _F), lambda i: (i, 0)),
            pl.BlockSpec((_BN, 1), lambda i: (i, 0)),
        ],
        out_shape=[
            jax.ShapeDtypeStruct((_NPAD, _F), jnp.float32),
            jax.ShapeDtypeStruct((_NPAD, 1), jnp.float32),
        ],
    )(S, rdeg, b2d, p2d)


def _tc_pool_b(scores2d, alive2d, kkeep):
    def body(sc_ref, alive_ref, out_ref):
        sc = sc_ref[...]
        bits = lax.bitcast_convert_type(sc, jnp.int32)
        ubits = lax.bitcast_convert_type(sc, jnp.uint32)
        flip = jnp.where(bits < 0, jnp.uint32(0xFFFFFFFF),
                         jnp.uint32(0x80000000))
        keys = jnp.where(alive_ref[...] > 0.5, ubits ^ flip, jnp.uint32(0))

        def bit(t, T):
            cand = T | (jnp.uint32(1) << (jnp.uint32(31) - t.astype(jnp.uint32)))
            cnt = jnp.sum((keys >= cand).astype(jnp.int32))
            return jnp.where(cnt >= kkeep, cand, T)

        T = lax.fori_loop(0, 32, bit, jnp.uint32(0))
        out_ref[...] = (keys >= T).astype(jnp.float32)

    return pl.pallas_call(
        body,
        out_shape=jax.ShapeDtypeStruct((_NROW, 128), jnp.float32),
    )(scores2d, alive2d)


def _tc_final(S, rdeg, b2d, alive_col, WoutP, boutP):
    def body(s_ref, rdeg_ref, b_ref, alive_ref, wo_ref, bo_ref, out_ref):
        def blk(i, g):
            sl = pl.ds(i * _BN, _BN)
            st = s_ref[0, sl, :] + s_ref[1, sl, :]
            h = jnp.maximum(st * rdeg_ref[sl, :] + b_ref[...], 0.0)
            return g + jnp.sum(h * alive_ref[sl, :], axis=0, keepdims=True)

        g = lax.fori_loop(0, _NB, blk, jnp.zeros((1, _F), jnp.float32))
        g = g * (1.0 / 2500.0)
        z = jnp.dot(g, wo_ref[...], preferred_element_type=jnp.float32)
        z = z + bo_ref[...]
        col = lax.broadcasted_iota(jnp.int32, (1, _F), 1)
        z = jnp.where(col < 2, z, -1e30)
        m = jnp.max(z)
        e = jnp.exp(z - m)
        out_ref[...] = e / jnp.sum(e)

    return pl.pallas_call(
        body,
        out_shape=jax.ShapeDtypeStruct((1, _F), jnp.float32),
    )(S, rdeg, b2d, alive_col, WoutP, boutP)


def kernel(x, edge_index, W1, b1, p1, W2, b2, p2, W3, b3, Wout, bout):
    f32 = jnp.float32
    src = edge_index[0]
    dst = edge_index[1]
    npadrows = _NPAD - _NN
    padi = _NN + (jnp.arange(_EPAD - _EE, dtype=jnp.int32) % npadrows)
    src2d = jnp.concatenate([src, padi]).reshape(_EROWS, 128)
    dst2d = jnp.concatenate([dst, padi]).reshape(_EROWS, 128)
    xp = jnp.pad(x, ((0, npadrows), (0, 0)))
    alive = (jnp.arange(_NPAD, dtype=jnp.int32) < _NN).astype(f32)
    alive = alive.reshape(_NROW, 128)
    WoutP = jnp.pad(Wout, ((0, 0), (0, _F - Wout.shape[1])))
    boutP = jnp.pad(bout, (0, _F - bout.shape[0])).reshape(1, _F)

    xg = xp
    esrc, edst, ecnt = src2d, dst2d, None
    for (W, b, p, kkeep) in ((W1, b1, p1, _NN // 2), (W2, b2, p2, _NN // 4)):
        y = _tc_matmul(xg, W)
        degp, csrc, cdst, ecnt = _sc_degc(esrc, edst, ecnt,
                                          alive.reshape(_NPAD))
        esrc = csrc.reshape(_NW * _CAP, 128)
        edst = cdst.reshape(_NW * _CAP, 128)
        y2, rdeg = _tc_scale(y, degp.reshape(_NC, _NPAD, 1),
                             alive.reshape(_NPAD, 1))
        S = _sc_agg(esrc, edst, ecnt, y2)
        xg, scores = _tc_pool_a(S, rdeg, b.reshape(1, _F), p.reshape(1, _F))
        alive = _tc_pool_b(scores.reshape(_NROW, 128), alive, kkeep)

    y = _tc_matmul(xg, W3)
    degp, csrc, cdst, ecnt = _sc_degc(esrc, edst, ecnt, alive.reshape(_NPAD))
    esrc = csrc.reshape(_NW * _CAP, 128)
    edst = cdst.reshape(_NW * _CAP, 128)
    y2, rdeg = _tc_scale(y, degp.reshape(_NC, _NPAD, 1),
                         alive.reshape(_NPAD, 1))
    S = _sc_agg(esrc, edst, ecnt, y2)
    probs = _tc_final(S, rdeg, b3.reshape(1, _F), alive.reshape(_NPAD, 1),
                      WoutP, boutP)
    return probs[0, :2]


# speedup vs baseline: 51.3933x; 1.0145x over previous
import functools

import jax
import jax.numpy as jnp
from jax import lax
from jax.experimental import pallas as pl
from jax.experimental.pallas import tpu as pltpu
from jax.experimental.pallas import tpu_sc as plsc

_NC, _NS = 2, 16
_NW = _NC * _NS
_NN = 10000
_NPAD = 10240
_NROW = _NPAD // 128
_EE = 320000
_EPAD = 327680
_EROWS = _EPAD // 128
_RPT = _EROWS // _NW
_TSL = _NPAD // _NS
_CAP = 88
_CAPW = _CAP * 128
_F = 128
_BN = 1024
_NB = _NPAD // _BN


def _sc_mesh():
    return plsc.VectorSubcoreMesh(
        core_axis_name="c", subcore_axis_name="s",
        num_cores=_NC, num_subcores=_NS)


def _sc_degc(esrc, edst, ecnt, alive_flat):
    rin = esrc.shape[0] // _NW
    has_cnt = ecnt is not None

    def k(*args):
        if has_cnt:
            (esrc_hbm, edst_hbm, ecnt_hbm, alive_hbm, deg_hbm, csrc_hbm,
             cdst_hbm, ccnt_hbm, srcv, dstv, av, wv, csrcv, cdstv, cntv,
             acc) = args
        else:
            (esrc_hbm, edst_hbm, alive_hbm, deg_hbm, csrc_hbm,
             cdst_hbm, ccnt_hbm, srcv, dstv, av, wv, csrcv, cdstv, cntv,
             acc) = args
        c = lax.axis_index("c")
        s = lax.axis_index("s")
        w = s * _NC + c
        base = s * _TSL
        for q in range(8):
            wv[pl.ds(q * 16, 16)] = jnp.zeros((16,), jnp.float32)
        for q in range(_TSL // 128):
            pltpu.sync_copy(wv, acc.at[pl.ds(base + q * 128, 128)])
        if has_cnt:
            pltpu.sync_copy(alive_hbm, av)
        pltpu.sync_copy(esrc_hbm.at[pl.ds(w * rin, rin)], srcv)
        pltpu.sync_copy(edst_hbm.at[pl.ds(w * rin, rin)], dstv)
        if has_cnt:
            pltpu.sync_copy(ecnt_hbm.at[w], cntv)
        plsc.subcore_barrier()
        cin = cntv[0, pl.ds(0, 16)][0] if has_cnt else rin

        def body(j, cur):
            for q in range(8):
                sv = srcv[j, pl.ds(q * 16, 16)]
                dv = dstv[j, pl.ds(q * 16, 16)]
                if has_cnt:
                    a = (plsc.load_gather(av, [sv])
                         * plsc.load_gather(av, [dv]))
                else:
                    a = ((sv < _NN) & (dv < _NN)).astype(jnp.float32)
                wv[pl.ds(q * 16, 16)] = a
                m = a > 0.0
                plsc.store_compressed(csrcv.at[pl.ds(cur, 16)], sv, mask=m)
                plsc.store_compressed(cdstv.at[pl.ds(cur, 16)], dv, mask=m)
                cur = cur + jnp.sum(m.astype(jnp.int32))
            pltpu.sync_copy(wv, acc.at[dstv.at[j]], add=True)
            return cur

        cur = lax.fori_loop(0, cin, body, jnp.int32(0))
        ii = lax.iota(jnp.int32, 16)
        for q in range(8):
            pv = _NN + ((ii * 8 + q) % (_NPAD - _NN))
            csrcv[pl.ds(cur + q * 16, 16)] = pv
            cdstv[pl.ds(cur + q * 16, 16)] = pv
        nch = (cur + 127) // 128
        cntv[0, :] = jnp.zeros((16,), jnp.int32) + nch
        pltpu.sync_copy(cntv, ccnt_hbm.at[w])
        pltpu.sync_copy(csrcv, csrc_hbm.at[pl.ds(w * _CAPW, _CAPW)])
        pltpu.sync_copy(cdstv, cdst_hbm.at[pl.ds(w * _CAPW, _CAPW)])
        plsc.subcore_barrier()
        pltpu.sync_copy(acc.at[pl.ds(base, _TSL)],
                        deg_hbm.at[c, pl.ds(base, _TSL)])

    kk = functools.partial(
        pl.kernel,
        out_type=[
            jax.ShapeDtypeStruct((_NC, _NPAD), jnp.float32),
            jax.ShapeDtypeStruct((_NW * _CAPW,), jnp.int32),
            jax.ShapeDtypeStruct((_NW * _CAPW,), jnp.int32),
            jax.ShapeDtypeStruct((_NW, 1, 16), jnp.int32),
        ],
        mesh=_sc_mesh(),
        compiler_params=pltpu.CompilerParams(needs_layout_passes=False),
        scratch_types=[
            pltpu.VMEM((rin, 128), jnp.int32),
            pltpu.VMEM((rin, 128), jnp.int32),
            pltpu.VMEM((_NPAD,), jnp.float32),
            pltpu.VMEM((128,), jnp.float32),
            pltpu.VMEM((_CAPW,), jnp.int32),
            pltpu.VMEM((_CAPW,), jnp.int32),
            pltpu.VMEM((1, 16), jnp.int32),
            pltpu.VMEM_SHARED((_NPAD,), jnp.float32),
        ],
    )(k)
    if has_cnt:
        return kk(esrc, edst, ecnt, alive_flat)
    return kk(esrc, edst, alive_flat)


def _sc_agg(src2d, dst2d, ccnt, y2):
    @functools.partial(
        pl.kernel,
        out_type=jax.ShapeDtypeStruct((_NC, _NPAD, _F), jnp.float32),
        mesh=_sc_mesh(),
        compiler_params=pltpu.CompilerParams(needs_layout_passes=False),
        scratch_types=[
            pltpu.VMEM((_CAP, 128), jnp.int32),
            pltpu.VMEM((_CAP, 128), jnp.int32),
            pltpu.VMEM((1, 16), jnp.int32),
            pltpu.VMEM((128, _F), jnp.float32),
            pltpu.VMEM_SHARED((_NPAD, _F), jnp.float32),
        ],
    )
    def k(src_hbm, dst_hbm, ccnt_hbm, y2_hbm, out_hbm, srcv, dstv,
          cntv, rows, acc):
        c = lax.axis_index("c")
        s = lax.axis_index("s")
        w = s * _NC + c
        base = s * _TSL

        @pl.when(c == 0)
        def _():
            pltpu.sync_copy(y2_hbm.at[pl.ds(base, _TSL)],
                            acc.at[pl.ds(base, _TSL)])

        @pl.when(c != 0)
        def _():
            def zbody(r, carry):
                for q in range(8):
                    rows[r, pl.ds(q * 16, 16)] = jnp.zeros((16,),
                                                           jnp.float32)
                return carry

            lax.fori_loop(0, 128, zbody, 0)
            for t in range(_TSL // 128):
                pltpu.sync_copy(rows, acc.at[pl.ds(base + t * 128, 128)])

        plsc.subcore_barrier()


        def fire_s(j, b):
            pltpu.async_copy(rows.at[b], acc.at[dstv.at[j]], sems[b],
                             add=True)

        def wait_s(b):
            pltpu.make_async_copy(rows.at[b], acc.at[dstv.at[0]],
                                  sems[b]).wait()

        pltpu.sync_copy(src_hbm.at[pl.ds(w * _CAP, _CAP)], srcv)
        pltpu.sync_copy(dst_hbm.at[pl.ds(w * _CAP, _CAP)], dstv)
        pltpu.sync_copy(ccnt_hbm.at[w], cntv)

        def body(j, carry):
            pltpu.sync_copy(y2_hbm.at[srcv.at[j]], rows)
            pltpu.sync_copy(rows, acc.at[dstv.at[j]], add=True)
            return carry

        lax.fori_loop(0, cntv[0, pl.ds(0, 16)][0], body, 0)
        plsc.subcore_barrier()
        pltpu.sync_copy(acc.at[pl.ds(base, _TSL)],
                        out_hbm.at[c, pl.ds(base, _TSL)])

    return k(src2d, dst2d, ccnt, y2)


def _tc_matmul(xg, W):
    def body(x_ref, w_ref, y_ref):
        y_ref[...] = jnp.dot(x_ref[...], w_ref[...],
                             preferred_element_type=jnp.float32)

    return pl.pallas_call(
        body,
        grid=(_NB,),
        in_specs=[
            pl.BlockSpec((_BN, _F), lambda i: (i, 0)),
            pl.BlockSpec((_F, _F), lambda i: (0, 0)),
        ],
        out_specs=pl.BlockSpec((_BN, _F), lambda i: (i, 0)),
        out_shape=jax.ShapeDtypeStruct((_NPAD, _F), jnp.float32),
    )(xg, W)


def _tc_scale(y, degp, alive_col):
    def body(y_ref, deg_ref, alive_ref, y2_ref, rdeg_ref):
        d = deg_ref[...]
        rdeg = lax.rsqrt(d[0] + d[1] + 1.0)
        y2_ref[...] = y_ref[...] * (rdeg * alive_ref[...])
        rdeg_ref[...] = rdeg

    return pl.pallas_call(
        body,
        grid=(_NB,),
        in_specs=[
            pl.BlockSpec((_BN, _F), lambda i: (i, 0)),
            pl.BlockSpec((_NC, _BN, 1), lambda i: (0, i, 0)),
            pl.BlockSpec((_BN, 1), lambda i: (i, 0)),
        ],
        out_specs=[
            pl.BlockSpec((_BN, _F), lambda i: (i, 0)),
            pl.BlockSpec((_BN, 1), lambda i: (i, 0)),
        ],
        out_shape=[
            jax.ShapeDtypeStruct((_NPAD, _F), jnp.float32),
            jax.ShapeDtypeStruct((_NPAD, 1), jnp.float32),
        ],
    )(y, degp, alive_col)


def _tc_pool_a(S, rdeg, b2d, p2d):
    def body(s_ref, rdeg_ref, b_ref, p_ref, xg_ref, sc_ref):
        p = p_ref[...]
        pn = lax.rsqrt(jnp.sum(p * p))
        st = s_ref[...]
        h = jnp.maximum((st[0] + st[1]) * rdeg_ref[...] + b_ref[...], 0.0)
        sc = jnp.sum(h * p, axis=1, keepdims=True) * pn
        xg_ref[...] = h * jnp.tanh(sc)
        sc_ref[...] = sc

    return pl.pallas_call(
        body,
        grid=(_NB,),
        in_specs=[
            pl.BlockSpec((_NC, _BN, _F), lambda i: (0, i, 0)),
            pl.BlockSpec((_BN, 1), lambda i: (i, 0)),
            pl.BlockSpec((1, _F), lambda i: (0, 0)),
            pl.BlockSpec((1, _F), lambda i: (0, 0)),
        ],
        out_specs=[
            pl.BlockSpec((_BN, _F), lambda i: (i, 0)),
            pl.BlockSpec((_BN, 1), lambda i: (i, 0)),
        ],
        out_shape=[
            jax.ShapeDtypeStruct((_NPAD, _F), jnp.float32),
            jax.ShapeDtypeStruct((_NPAD, 1), jnp.float32),
        ],
    )(S, rdeg, b2d, p2d)


def _tc_pool_b(scores2d, alive2d, kkeep):
    def body(sc_ref, alive_ref, out_ref):
        sc = sc_ref[...]
        bits = lax.bitcast_convert_type(sc, jnp.int32)
        ubits = lax.bitcast_convert_type(sc, jnp.uint32)
        flip = jnp.where(bits < 0, jnp.uint32(0xFFFFFFFF),
                         jnp.uint32(0x80000000))
        keys = jnp.where(alive_ref[...] > 0.5, ubits ^ flip, jnp.uint32(0))

        def bit(t, T):
            cand = T | (jnp.uint32(1) << (jnp.uint32(31) - t.astype(jnp.uint32)))
            cnt = jnp.sum((keys >= cand).astype(jnp.int32))
            return jnp.where(cnt >= kkeep, cand, T)

        T = lax.fori_loop(0, 32, bit, jnp.uint32(0))
        out_ref[...] = (keys >= T).astype(jnp.float32)

    return pl.pallas_call(
        body,
        out_shape=jax.ShapeDtypeStruct((_NROW, 128), jnp.float32),
    )(scores2d, alive2d)


def _tc_final(S, rdeg, b2d, alive_col, WoutP, boutP):
    def body(s_ref, rdeg_ref, b_ref, alive_ref, wo_ref, bo_ref, out_ref):
        def blk(i, g):
            sl = pl.ds(i * _BN, _BN)
            st = s_ref[0, sl, :] + s_ref[1, sl, :]
            h = jnp.maximum(st * rdeg_ref[sl, :] + b_ref[...], 0.0)
            return g + jnp.sum(h * alive_ref[sl, :], axis=0, keepdims=True)

        g = lax.fori_loop(0, _NB, blk, jnp.zeros((1, _F), jnp.float32))
        g = g * (1.0 / 2500.0)
        z = jnp.dot(g, wo_ref[...], preferred_element_type=jnp.float32)
        z = z + bo_ref[...]
        col = lax.broadcasted_iota(jnp.int32, (1, _F), 1)
        z = jnp.where(col < 2, z, -1e30)
        m = jnp.max(z)
        e = jnp.exp(z - m)
        out_ref[...] = e / jnp.sum(e)

    return pl.pallas_call(
        body,
        out_shape=jax.ShapeDtypeStruct((1, _F), jnp.float32),
    )(S, rdeg, b2d, alive_col, WoutP, boutP)


def kernel(x, edge_index, W1, b1, p1, W2, b2, p2, W3, b3, Wout, bout):
    f32 = jnp.float32
    src = edge_index[0]
    dst = edge_index[1]
    npadrows = _NPAD - _NN
    padi = _NN + (jnp.arange(_EPAD - _EE, dtype=jnp.int32) % npadrows)
    src2d = jnp.concatenate([src, padi]).reshape(_EROWS, 128)
    dst2d = jnp.concatenate([dst, padi]).reshape(_EROWS, 128)
    xp = jnp.pad(x, ((0, npadrows), (0, 0)))
    alive = (jnp.arange(_NPAD, dtype=jnp.int32) < _NN).astype(f32)
    alive = alive.reshape(_NROW, 128)
    WoutP = jnp.pad(Wout, ((0, 0), (0, _F - Wout.shape[1])))
    boutP = jnp.pad(bout, (0, _F - bout.shape[0])).reshape(1, _F)

    xg = xp
    esrc, edst, ecnt = src2d, dst2d, None
    for (W, b, p, kkeep) in ((W1, b1, p1, _NN // 2), (W2, b2, p2, _NN // 4)):
        y = _tc_matmul(xg, W)
        degp, csrc, cdst, ecnt = _sc_degc(esrc, edst, ecnt,
                                          alive.reshape(_NPAD))
        esrc = csrc.reshape(_NW * _CAP, 128)
        edst = cdst.reshape(_NW * _CAP, 128)
        y2, rdeg = _tc_scale(y, degp.reshape(_NC, _NPAD, 1),
                             alive.reshape(_NPAD, 1))
        S = _sc_agg(esrc, edst, ecnt, y2)
        xg, scores = _tc_pool_a(S, rdeg, b.reshape(1, _F), p.reshape(1, _F))
        alive = _tc_pool_b(scores.reshape(_NROW, 128), alive, kkeep)

    y = _tc_matmul(xg, W3)
    degp, csrc, cdst, ecnt = _sc_degc(esrc, edst, ecnt, alive.reshape(_NPAD))
    esrc = csrc.reshape(_NW * _CAP, 128)
    edst = cdst.reshape(_NW * _CAP, 128)
    y2, rdeg = _tc_scale(y, degp.reshape(_NC, _NPAD, 1),
                         alive.reshape(_NPAD, 1))
    S = _sc_agg(esrc, edst, ecnt, y2)
    probs = _tc_final(S, rdeg, b3.reshape(1, _F), alive.reshape(_NPAD, 1),
                      WoutP, boutP)
    return probs[0, :2]


# speedup vs baseline: 51.5382x; 1.0028x over previous
import functools

import jax
import jax.numpy as jnp
from jax import lax
from jax.experimental import pallas as pl
from jax.experimental.pallas import tpu as pltpu
from jax.experimental.pallas import tpu_sc as plsc

_NC, _NS = 2, 16
_NW = _NC * _NS
_NN = 10000
_NPAD = 10240
_NROW = _NPAD // 128
_EE = 320000
_EPAD = 327680
_EROWS = _EPAD // 128
_RPT = _EROWS // _NW
_TSL = _NPAD // _NS
_CAP = 88
_CAPW = _CAP * 128
_F = 128
_BN = 1024
_NB = _NPAD // _BN


def _sc_mesh():
    return plsc.VectorSubcoreMesh(
        core_axis_name="c", subcore_axis_name="s",
        num_cores=_NC, num_subcores=_NS)


def _sc_degc(esrc, edst, ecnt, alive_flat):
    rin = esrc.shape[0] // _NW
    has_cnt = ecnt is not None

    def k(*args):
        if has_cnt:
            (esrc_hbm, edst_hbm, ecnt_hbm, alive_hbm, deg_hbm, csrc_hbm,
             cdst_hbm, ccnt_hbm, srcv, dstv, av, wv, csrcv, cdstv, cntv,
             acc) = args
        else:
            (esrc_hbm, edst_hbm, alive_hbm, deg_hbm, csrc_hbm,
             cdst_hbm, ccnt_hbm, srcv, dstv, av, wv, csrcv, cdstv, cntv,
             acc) = args
        c = lax.axis_index("c")
        s = lax.axis_index("s")
        w = s * _NC + c
        base = s * _TSL
        for q in range(8):
            wv[pl.ds(q * 16, 16)] = jnp.zeros((16,), jnp.float32)
        for q in range(_TSL // 128):
            pltpu.sync_copy(wv, acc.at[pl.ds(base + q * 128, 128)])
        if has_cnt:
            pltpu.sync_copy(alive_hbm, av)
        pltpu.sync_copy(esrc_hbm.at[pl.ds(w * rin, rin)], srcv)
        pltpu.sync_copy(edst_hbm.at[pl.ds(w * rin, rin)], dstv)
        if has_cnt:
            pltpu.sync_copy(ecnt_hbm.at[w], cntv)
        plsc.subcore_barrier()
        cin = cntv[0, pl.ds(0, 16)][0] if has_cnt else rin

        def body(j, cur):
            for q in range(8):
                sv = srcv[j, pl.ds(q * 16, 16)]
                dv = dstv[j, pl.ds(q * 16, 16)]
                if has_cnt:
                    a = (plsc.load_gather(av, [sv])
                         * plsc.load_gather(av, [dv]))
                else:
                    a = ((sv < _NN) & (dv < _NN)).astype(jnp.float32)
                wv[pl.ds(q * 16, 16)] = a
                m = a > 0.0
                plsc.store_compressed(csrcv.at[pl.ds(cur, 16)], sv, mask=m)
                plsc.store_compressed(cdstv.at[pl.ds(cur, 16)], dv, mask=m)
                cur = cur + jnp.sum(m.astype(jnp.int32))
            pltpu.sync_copy(wv, acc.at[dstv.at[j]], add=True)
            return cur

        cur = lax.fori_loop(0, cin, body, jnp.int32(0))
        ii = lax.iota(jnp.int32, 16)
        for q in range(8):
            pv = _NN + ((ii * 8 + q) % (_NPAD - _NN))
            csrcv[pl.ds(cur + q * 16, 16)] = pv
            cdstv[pl.ds(cur + q * 16, 16)] = pv
        nch = (cur + 127) // 128
        cntv[0, :] = jnp.zeros((16,), jnp.int32) + nch
        pltpu.sync_copy(cntv, ccnt_hbm.at[w])
        pltpu.sync_copy(csrcv, csrc_hbm.at[pl.ds(w * _CAPW, _CAPW)])
        pltpu.sync_copy(cdstv, cdst_hbm.at[pl.ds(w * _CAPW, _CAPW)])
        plsc.subcore_barrier()
        pltpu.sync_copy(acc.at[pl.ds(base, _TSL)],
                        deg_hbm.at[c, pl.ds(base, _TSL)])

    kk = functools.partial(
        pl.kernel,
        out_type=[
            jax.ShapeDtypeStruct((_NC, _NPAD), jnp.float32),
            jax.ShapeDtypeStruct((_NW * _CAPW,), jnp.int32),
            jax.ShapeDtypeStruct((_NW * _CAPW,), jnp.int32),
            jax.ShapeDtypeStruct((_NW, 1, 16), jnp.int32),
        ],
        mesh=_sc_mesh(),
        compiler_params=pltpu.CompilerParams(needs_layout_passes=False),
        scratch_types=[
            pltpu.VMEM((rin, 128), jnp.int32),
            pltpu.VMEM((rin, 128), jnp.int32),
            pltpu.VMEM((_NPAD,), jnp.float32),
            pltpu.VMEM((128,), jnp.float32),
            pltpu.VMEM((_CAPW,), jnp.int32),
            pltpu.VMEM((_CAPW,), jnp.int32),
            pltpu.VMEM((1, 16), jnp.int32),
            pltpu.VMEM_SHARED((_NPAD,), jnp.float32),
        ],
    )(k)
    if has_cnt:
        return kk(esrc, edst, ecnt, alive_flat)
    return kk(esrc, edst, alive_flat)


def _sc_agg(src2d, dst2d, ccnt, y2):
    @functools.partial(
        pl.kernel,
        out_type=jax.ShapeDtypeStruct((_NC, _NPAD, _F), jnp.float32),
        mesh=_sc_mesh(),
        compiler_params=pltpu.CompilerParams(needs_layout_passes=False),
        scratch_types=[
            pltpu.VMEM((_CAP, 128), jnp.int32),
            pltpu.VMEM((_CAP, 128), jnp.int32),
            pltpu.VMEM((1, 16), jnp.int32),
            pltpu.VMEM((128, _F), jnp.float32),
            pltpu.VMEM_SHARED((_NPAD, _F), jnp.float32),
        ],
    )
    def k(src_hbm, dst_hbm, ccnt_hbm, y2_hbm, out_hbm, srcv, dstv,
          cntv, rows, acc):
        c = lax.axis_index("c")
        s = lax.axis_index("s")
        w = s * _NC + c
        base = s * _TSL

        @pl.when(c == 0)
        def _():
            pltpu.sync_copy(y2_hbm.at[pl.ds(base, _TSL)],
                            acc.at[pl.ds(base, _TSL)])

        @pl.when(c != 0)
        def _():
            def zbody(r, carry):
                for q in range(8):
                    rows[r, pl.ds(q * 16, 16)] = jnp.zeros((16,),
                                                           jnp.float32)
                return carry

            lax.fori_loop(0, 128, zbody, 0)
            for t in range(_TSL // 128):
                pltpu.sync_copy(rows, acc.at[pl.ds(base + t * 128, 128)])

        plsc.subcore_barrier()


        def fire_s(j, b):
            pltpu.async_copy(rows.at[b], acc.at[dstv.at[j]], sems[b],
                             add=True)

        def wait_s(b):
            pltpu.make_async_copy(rows.at[b], acc.at[dstv.at[0]],
                                  sems[b]).wait()

        pltpu.sync_copy(src_hbm.at[pl.ds(w * _CAP, _CAP)], srcv)
        pltpu.sync_copy(dst_hbm.at[pl.ds(w * _CAP, _CAP)], dstv)
        pltpu.sync_copy(ccnt_hbm.at[w], cntv)

        def body(j, carry):
            pltpu.sync_copy(y2_hbm.at[srcv.at[j]], rows)
            pltpu.sync_copy(rows, acc.at[dstv.at[j]], add=True)
            return carry

        lax.fori_loop(0, cntv[0, pl.ds(0, 16)][0], body, 0)
        plsc.subcore_barrier()
        pltpu.sync_copy(acc.at[pl.ds(base, _TSL)],
                        out_hbm.at[c, pl.ds(base, _TSL)])

    return k(src2d, dst2d, ccnt, y2)


def _tc_matmul(xg, W):
    def body(x_ref, w_ref, y_ref):
        y_ref[...] = jnp.dot(x_ref[...], w_ref[...],
                             preferred_element_type=jnp.float32)

    return pl.pallas_call(
        body,
        grid=(_NB,),
        in_specs=[
            pl.BlockSpec((_BN, _F), lambda i: (i, 0)),
            pl.BlockSpec((_F, _F), lambda i: (0, 0)),
        ],
        out_specs=pl.BlockSpec((_BN, _F), lambda i: (i, 0)),
        out_shape=jax.ShapeDtypeStruct((_NPAD, _F), jnp.float32),
    )(xg, W)


def _tc_scale(y, degp, alive_col):
    def body(y_ref, deg_ref, alive_ref, y2_ref, rdeg_ref):
        d = deg_ref[...]
        rdeg = lax.rsqrt(d[0] + d[1] + 1.0)
        y2_ref[...] = y_ref[...] * (rdeg * alive_ref[...])
        rdeg_ref[...] = rdeg

    return pl.pallas_call(
        body,
        grid=(_NB,),
        in_specs=[
            pl.BlockSpec((_BN, _F), lambda i: (i, 0)),
            pl.BlockSpec((_NC, _BN, 1), lambda i: (0, i, 0)),
            pl.BlockSpec((_BN, 1), lambda i: (i, 0)),
        ],
        out_specs=[
            pl.BlockSpec((_BN, _F), lambda i: (i, 0)),
            pl.BlockSpec((_BN, 1), lambda i: (i, 0)),
        ],
        out_shape=[
            jax.ShapeDtypeStruct((_NPAD, _F), jnp.float32),
            jax.ShapeDtypeStruct((_NPAD, 1), jnp.float32),
        ],
    )(y, degp, alive_col)


def _tc_pool_a(S, rdeg, b2d, p2d, Wn):
    def body(s_ref, rdeg_ref, b_ref, p_ref, wn_ref, y_ref, sc_ref):
        p = p_ref[...]
        pn = lax.rsqrt(jnp.sum(p * p))
        st = s_ref[...]
        h = jnp.maximum((st[0] + st[1]) * rdeg_ref[...] + b_ref[...], 0.0)
        sc = jnp.sum(h * p, axis=1, keepdims=True) * pn
        y_ref[...] = jnp.dot(h * jnp.tanh(sc), wn_ref[...],
                             preferred_element_type=jnp.float32)
        sc_ref[...] = sc

    return pl.pallas_call(
        body,
        grid=(_NB,),
        in_specs=[
            pl.BlockSpec((_NC, _BN, _F), lambda i: (0, i, 0)),
            pl.BlockSpec((_BN, 1), lambda i: (i, 0)),
            pl.BlockSpec((1, _F), lambda i: (0, 0)),
            pl.BlockSpec((1, _F), lambda i: (0, 0)),
            pl.BlockSpec((_F, _F), lambda i: (0, 0)),
        ],
        out_specs=[
            pl.BlockSpec((_BN, _F), lambda i: (i, 0)),
            pl.BlockSpec((_BN, 1), lambda i: (i, 0)),
        ],
        out_shape=[
            jax.ShapeDtypeStruct((_NPAD, _F), jnp.float32),
            jax.ShapeDtypeStruct((_NPAD, 1), jnp.float32),
        ],
    )(S, rdeg, b2d, p2d, Wn)


def _tc_pool_b(scores2d, alive2d, kkeep):
    def body(sc_ref, alive_ref, out_ref):
        sc = sc_ref[...]
        bits = lax.bitcast_convert_type(sc, jnp.int32)
        ubits = lax.bitcast_convert_type(sc, jnp.uint32)
        flip = jnp.where(bits < 0, jnp.uint32(0xFFFFFFFF),
                         jnp.uint32(0x80000000))
        keys = jnp.where(alive_ref[...] > 0.5, ubits ^ flip, jnp.uint32(0))

        def bit(t, T):
            cand = T | (jnp.uint32(1) << (jnp.uint32(31) - t.astype(jnp.uint32)))
            cnt = jnp.sum((keys >= cand).astype(jnp.int32))
            return jnp.where(cnt >= kkeep, cand, T)

        T = lax.fori_loop(0, 32, bit, jnp.uint32(0))
        out_ref[...] = (keys >= T).astype(jnp.float32)

    return pl.pallas_call(
        body,
        out_shape=jax.ShapeDtypeStruct((_NROW, 128), jnp.float32),
    )(scores2d, alive2d)


def _tc_final(S, rdeg, b2d, alive_col, WoutP, boutP):
    def body(s_ref, rdeg_ref, b_ref, alive_ref, wo_ref, bo_ref, out_ref):
        def blk(i, g):
            sl = pl.ds(i * _BN, _BN)
            st = s_ref[0, sl, :] + s_ref[1, sl, :]
            h = jnp.maximum(st * rdeg_ref[sl, :] + b_ref[...], 0.0)
            return g + jnp.sum(h * alive_ref[sl, :], axis=0, keepdims=True)

        g = lax.fori_loop(0, _NB, blk, jnp.zeros((1, _F), jnp.float32))
        g = g * (1.0 / 2500.0)
        z = jnp.dot(g, wo_ref[...], preferred_element_type=jnp.float32)
        z = z + bo_ref[...]
        col = lax.broadcasted_iota(jnp.int32, (1, _F), 1)
        z = jnp.where(col < 2, z, -1e30)
        m = jnp.max(z)
        e = jnp.exp(z - m)
        out_ref[...] = e / jnp.sum(e)

    return pl.pallas_call(
        body,
        out_shape=jax.ShapeDtypeStruct((1, _F), jnp.float32),
    )(S, rdeg, b2d, alive_col, WoutP, boutP)


def kernel(x, edge_index, W1, b1, p1, W2, b2, p2, W3, b3, Wout, bout):
    f32 = jnp.float32
    src = edge_index[0]
    dst = edge_index[1]
    npadrows = _NPAD - _NN
    padi = _NN + (jnp.arange(_EPAD - _EE, dtype=jnp.int32) % npadrows)
    src2d = jnp.concatenate([src, padi]).reshape(_EROWS, 128)
    dst2d = jnp.concatenate([dst, padi]).reshape(_EROWS, 128)
    xp = jnp.pad(x, ((0, npadrows), (0, 0)))
    alive = (jnp.arange(_NPAD, dtype=jnp.int32) < _NN).astype(f32)
    alive = alive.reshape(_NROW, 128)
    WoutP = jnp.pad(Wout, ((0, 0), (0, _F - Wout.shape[1])))
    boutP = jnp.pad(bout, (0, _F - bout.shape[0])).reshape(1, _F)

    esrc, edst, ecnt = src2d, dst2d, None
    y = _tc_matmul(xp, W1)
    for (b, p, Wn, kkeep) in ((b1, p1, W2, _NN // 2), (b2, p2, W3, _NN // 4)):
        degp, csrc, cdst, ecnt = _sc_degc(esrc, edst, ecnt,
                                          alive.reshape(_NPAD))
        esrc = csrc.reshape(_NW * _CAP, 128)
        edst = cdst.reshape(_NW * _CAP, 128)
        y2, rdeg = _tc_scale(y, degp.reshape(_NC, _NPAD, 1),
                             alive.reshape(_NPAD, 1))
        S = _sc_agg(esrc, edst, ecnt, y2)
        y, scores = _tc_pool_a(S, rdeg, b.reshape(1, _F), p.reshape(1, _F),
                               Wn)
        alive = _tc_pool_b(scores.reshape(_NROW, 128), alive, kkeep)

    degp, csrc, cdst, ecnt = _sc_degc(esrc, edst, ecnt, alive.reshape(_NPAD))
    esrc = csrc.reshape(_NW * _CAP, 128)
    edst = cdst.reshape(_NW * _CAP, 128)
    y2, rdeg = _tc_scale(y, degp.reshape(_NC, _NPAD, 1),
                         alive.reshape(_NPAD, 1))
    S = _sc_agg(esrc, edst, ecnt, y2)
    probs = _tc_final(S, rdeg, b3.reshape(1, _F), alive.reshape(_NPAD, 1),
                      WoutP, boutP)
    return probs[0, :2]
